# baseline (speedup 1.0000x reference)
import functools

import jax
import jax.numpy as jnp
from jax import lax
from jax.experimental import pallas as pl
from jax.experimental.pallas import tpu as pltpu
from jax.experimental.pallas import tpu_sc as plsc

NC = 2
NS = 16
NW = NC * NS
L = 16


def _make_sc_kernel(B, D):
    b_per_w = B // NW
    mesh = plsc.VectorSubcoreMesh(core_axis_name="c", subcore_axis_name="s")

    @functools.partial(
        pl.kernel,
        mesh=mesh,
        out_type=jax.ShapeDtypeStruct((B, 2 * D), jnp.float32),
        scratch_types=[
            pltpu.VMEM((b_per_w,), jnp.int32),
            pltpu.VMEM((D,), jnp.float32),
            pltpu.VMEM((L, 8, D), jnp.float32),
            pltpu.VMEM((b_per_w, 2 * D), jnp.float32),
            pltpu.SemaphoreType.DMA,
            pltpu.SemaphoreType.DMA,
        ],
    )
    def k(idx_hbm, pos_hbm, table3_hbm, out_hbm,
          idx_v, pos_v, ring_v, comb_v, sem_t, sem_p):
        wid = lax.axis_index("s") * NC + lax.axis_index("c")
        base = wid * b_per_w
        for j in range(b_per_w // 128):
            pltpu.sync_copy(
                idx_hbm.at[pl.ds(base + j * 128, 128)],
                idx_v.at[pl.ds(j * 128, 128)])
        pltpu.async_copy(pos_hbm, pos_v, sem_p).wait()
        pvals = [pos_v[pl.ds(c * L, L)] for c in range(D // L)]

        def group(g, _):
            iv = idx_v[pl.ds(g * L, L)]
            cps = []
            subs = []
            for l in range(L):
                s = iv[l]
                subs.append(s & 7)
                cps.append(pltpu.async_copy(
                    table3_hbm.at[s >> 3], ring_v.at[l], sem_t))
            for cp in cps:
                cp.wait()
            for l in range(L):
                for c in range(D // L):
                    comb_v[g * L + l, pl.ds(c * L, L)] = (
                        ring_v[l, subs[l], pl.ds(c * L, L)])
                for c in range(D // L):
                    comb_v[g * L + l, pl.ds(D + c * L, L)] = pvals[c]
            return 0

        lax.fori_loop(0, b_per_w // L, group, 0)
        pltpu.sync_copy(comb_v, out_hbm.at[pl.ds(base, b_per_w)])

    return k


def kernel(input, input_position, table, pos_enc):
    B = input.shape[0]
    D = table.shape[1]
    idx = input.astype(jnp.int32)
    table3 = table.reshape(table.shape[0] // 8, 8, D)
    pos_row = pos_enc[input_position]
    k = _make_sc_kernel(B, D)
    return k(idx, pos_row, table3)


# speedup vs baseline: 1.0702x; 1.0702x over previous
import functools

import jax
import jax.numpy as jnp
from jax import lax
from jax.experimental import pallas as pl
from jax.experimental.pallas import tpu as pltpu
from jax.experimental.pallas import tpu_sc as plsc

NC = 2
NS = 16
NW = NC * NS
L = 16


def _make_sc_kernel(B, D):
    b_per_w = B // NW
    mesh = plsc.VectorSubcoreMesh(core_axis_name="c", subcore_axis_name="s")

    @functools.partial(
        pl.kernel,
        mesh=mesh,
        out_type=jax.ShapeDtypeStruct((B, 2 * D), jnp.float32),
        scratch_types=[
            pltpu.VMEM((b_per_w,), jnp.int32),
            pltpu.VMEM((D,), jnp.float32),
            pltpu.VMEM((2, L, 8, D), jnp.float32),
            pltpu.VMEM((b_per_w, 2 * D), jnp.float32),
            pltpu.SemaphoreType.DMA,
            pltpu.SemaphoreType.DMA,
            pltpu.SemaphoreType.DMA,
        ],
    )
    def k(idx_hbm, pos_hbm, table3_hbm, out_hbm,
          idx_v, pos_v, ring_v, comb_v, sem0, sem1, sem_p):
        wid = lax.axis_index("s") * NC + lax.axis_index("c")
        base = wid * b_per_w
        NG = b_per_w // L
        for j in range(b_per_w // 128):
            pltpu.sync_copy(
                idx_hbm.at[pl.ds(base + j * 128, 128)],
                idx_v.at[pl.ds(j * 128, 128)])
        pltpu.async_copy(pos_hbm, pos_v, sem_p).wait()
        pvals = [pos_v[pl.ds(c * L, L)] for c in range(D // L)]

        def fire(g, slot, sem):
            iv = idx_v[pl.ds(g * L, L)]
            for l in range(L):
                pltpu.async_copy(
                    table3_hbm.at[iv[l] >> 3], ring_v.at[slot, l], sem)

        def drain(slot, sem):
            for l in range(L):
                pltpu.make_async_copy(
                    table3_hbm.at[0], ring_v.at[slot, l], sem).wait()

        def extract(g, slot):
            iv = idx_v[pl.ds(g * L, L)]
            for l in range(L):
                sub = iv[l] & 7
                for c in range(D // L):
                    comb_v[g * L + l, pl.ds(c * L, L)] = (
                        ring_v[slot, l, sub, pl.ds(c * L, L)])
                for c in range(D // L):
                    comb_v[g * L + l, pl.ds(D + c * L, L)] = pvals[c]

        fire(0, 0, sem0)

        def step(kk, _):
            g0 = 2 * kk
            fire(g0 + 1, 1, sem1)
            drain(0, sem0)
            extract(g0, 0)

            @pl.when(g0 + 2 < NG)
            def _():
                fire(g0 + 2, 0, sem0)

            drain(1, sem1)
            extract(g0 + 1, 1)
            return 0

        lax.fori_loop(0, NG // 2, step, 0)
        pltpu.sync_copy(comb_v, out_hbm.at[pl.ds(base, b_per_w)])

    return k


def kernel(input, input_position, table, pos_enc):
    B = input.shape[0]
    D = table.shape[1]
    idx = input.astype(jnp.int32)
    table3 = table.reshape(table.shape[0] // 8, 8, D)
    pos_row = pos_enc[input_position]
    k = _make_sc_kernel(B, D)
    return k(idx, pos_row, table3)
